# Initial kernel scaffold; baseline (speedup 1.0000x reference)
import jax
import jax.numpy as jnp
from jax.experimental import pallas as pl


def kernel(edge_index, edge_weight, user_emb, item_emb):
    raise NotImplementedError("write your pallas kernel here")


# speedup vs baseline: 1.1321x; 1.1321x over previous
import functools

import jax
import jax.numpy as jnp
from jax import lax
from jax.experimental import pallas as pl
from jax.experimental.pallas import tpu as pltpu
from jax.experimental.pallas import tpu_sc as plsc

USER_N = 5000
ITEM_N = 5000
N = USER_N + ITEM_N
D = 256
E = 160000
NT = 32
HALF = N // 2
RPT = 320
LPAD = 16 * RPT
NPAD = NT * RPT
SHIFT = LPAD - HALF
DV = D // 16
PKBITS = 14
PKMASK = (1 << PKBITS) - 1
CH = 5120
EPAD = NT * CH
EB = 512
CAP = 448
SEG = NT * CAP
MCAP = 6656
GB = 64
ORECIP = 6554
OSH = 21


def _partition_body(pk_hbm, w_hbm, rec_hbm, rw_hbm, cnt_hbm, pb, wb, recb, offb, cv, cvx):
    wid = lax.axis_index("s") * 2 + lax.axis_index("c")
    iota = lax.iota(jnp.int32, 16)
    zi = jnp.zeros((16,), jnp.int32)
    lane_masks = [iota == e for e in range(16)]

    cv[pl.ds(0, 16)] = zi
    cv[pl.ds(16, 16)] = zi

    def _bt(bt, _):
        pltpu.sync_copy(pk_hbm.at[wid * (CH // EB) + bt], pb)
        pltpu.sync_copy(w_hbm.at[wid * (CH // EB) + bt], wb)

        def _grp(g, _):
            sl = pl.ds(g * 16, 16)
            pv = pb[sl]
            sv = pv & PKMASK
            dv = lax.shift_right_logical(pv, PKBITS)
            sp = sv + jnp.where(sv >= HALF, SHIFT, 0)
            pd = dv + jnp.where(dv >= HALF, SHIFT, 0)
            o16 = lax.shift_right_logical(pd * ORECIP, OSH)
            lr16 = pd - o16 * RPT
            rec16 = sp | (lr16 << PKBITS)

            rankv = zi
            basev = zi
            hist_lo = zi
            hist_hi = zi
            for e in range(16):
                o_sc = o16[e]
                base_e = cv[pl.ds(o_sc, 16)][0]
                bo = o_sc + zi
                rankv = rankv + jnp.where((o16 == bo) & (iota > e), 1, 0)
                hist_lo = hist_lo + jnp.where(iota == bo, 1, 0)
                hist_hi = hist_hi + jnp.where(iota == (bo - 16), 1, 0)
                basev = jnp.where(lane_masks[e], base_e + zi, basev)
            cv[pl.ds(0, 16)] = cv[pl.ds(0, 16)] + hist_lo
            cv[pl.ds(16, 16)] = cv[pl.ds(16, 16)] + hist_hi

            pos = jnp.minimum(basev + rankv, CAP - 1)
            off16 = wid * SEG + o16 * CAP + pos
            recb[pl.ds(g * 16, 16)] = rec16
            r = g >> 3
            offb[r, pl.ds((g & 7) * 16, 16)] = off16
            return 0

        lax.fori_loop(0, EB // 16, _grp, 0)

        for kb in range(EB // 128):
            pltpu.sync_copy(recb.at[pl.ds(kb * 128, 128)], rec_hbm.at[offb.at[kb]])
            pltpu.sync_copy(wb.at[pl.ds(kb * 128, 128)], rw_hbm.at[offb.at[kb]])
        return 0

    lax.fori_loop(0, CH // EB, _bt, 0)

    cvx[pl.ds(0, 16)] = jnp.minimum(cv[pl.ds(0, 16)], CAP)
    cvx[pl.ds(16, 16)] = jnp.minimum(cv[pl.ds(16, 16)], CAP)
    pltpu.sync_copy(cvx, cnt_hbm.at[wid])


def _layer_body(final, rec_hbm, rw_hbm, cntt_hbm, ego_hbm, *rest):
    if final:
        e0_hbm, e1_hbm, out_hbm, cv, cvs, str_, stw, frec, fw, gidx, lrb, wvb, rows, acc = rest
    else:
        out_hbm, cv, cvs, str_, stw, frec, fw, gidx, lrb, wvb, rows, acc = rest

    wid = lax.axis_index("s") * 2 + lax.axis_index("c")
    iota = lax.iota(jnp.int32, 16)
    zf = jnp.zeros((16,), jnp.float32)
    zi = jnp.zeros((16,), jnp.int32)

    def _zr(r, _):
        for d in range(DV):
            acc[r, pl.ds(d * 16, 16)] = zf
        return 0

    lax.fori_loop(0, RPT, _zr, 0)

    pltpu.sync_copy(cntt_hbm.at[wid], cvs)
    cv[pl.ds(0, 16)] = cvs[pl.ds(0, 16)]
    cv[pl.ds(16, 16)] = cvs[pl.ds(16, 16)]

    def _seg(st, ptr):
        soff = st * SEG + wid * CAP
        pltpu.sync_copy(rec_hbm.at[pl.ds(soff, CAP)], str_)
        pltpu.sync_copy(rw_hbm.at[pl.ds(soff, CAP)], stw)
        n = cv[pl.ds(st, 16)][0]
        ng = (n + 15) >> 4

        def _cp(g, _):
            sl = pl.ds(g * 16, 16)
            valid = (g * 16 + iota) < n
            dsl = pl.ds(ptr + g * 16, 16)
            frec[dsl] = str_[sl]
            fw[dsl] = jnp.where(valid, stw[sl], 0.0)
            return 0

        lax.fori_loop(0, ng, _cp, 0)
        return jnp.minimum(ptr + ng * 16, MCAP - CAP)

    m = lax.fori_loop(0, NT, _seg, jnp.int32(0))

    for k in range(GB // 16):
        fw[pl.ds(m + k * 16, 16)] = zf

    nbat = (m + GB - 1) >> 6

    def _bat(b, _):
        base = b * GB

        def _unp(g, _):
            sl = pl.ds(base + g * 16, 16)
            pv = frec[sl]
            gidx[pl.ds(g * 16, 16)] = jnp.minimum(pv & PKMASK, NPAD - 1)
            lrb[pl.ds(g * 16, 16)] = jnp.minimum(
                lax.shift_right_logical(pv, PKBITS), RPT - 1)
            wvb[pl.ds(g * 16, 16)] = fw[sl]
            return 0

        lax.fori_loop(0, GB // 16, _unp, 0)
        pltpu.sync_copy(ego_hbm.at[gidx], rows)

        def _accg(g, _):
            lr16 = lrb[pl.ds(g * 16, 16)]
            w16 = wvb[pl.ds(g * 16, 16)]
            for e in range(16):
                lr = lr16[e]
                we = w16[e]
                r = g * 16 + e
                for d in range(DV):
                    sl = pl.ds(d * 16, 16)
                    acc[lr, sl] = acc[lr, sl] + rows[r, sl] * we
            return 0

        lax.fori_loop(0, GB // 16, _accg, 0)
        return 0

    lax.fori_loop(0, nbat, _bat, 0)

    obase = wid * RPT
    if final:
        CK = GB
        for k in range(RPT // CK):
            off = k * CK
            for other in (e0_hbm, e1_hbm, ego_hbm):
                pltpu.sync_copy(other.at[pl.ds(obase + off, CK)], rows)

                def _add(r, _):
                    for d in range(DV):
                        sl = pl.ds(d * 16, 16)
                        acc[off + r, sl] = acc[off + r, sl] + rows[r, sl]
                    return 0

                lax.fori_loop(0, CK, _add, 0)

        def _scale(r, _):
            for d in range(DV):
                sl = pl.ds(d * 16, 16)
                acc[r, sl] = acc[r, sl] * 0.25
            return 0

        lax.fori_loop(0, RPT, _scale, 0)
    pltpu.sync_copy(acc, out_hbm.at[pl.ds(obase, RPT)])


_MESH = dict(core_axis_name="c", subcore_axis_name="s")


def _make_partition():
    return pl.kernel(
        _partition_body,
        out_type=(
            jax.ShapeDtypeStruct((NT * SEG,), jnp.int32),
            jax.ShapeDtypeStruct((NT * SEG,), jnp.float32),
            jax.ShapeDtypeStruct((NT, NT), jnp.int32),
        ),
        mesh=plsc.VectorSubcoreMesh(**_MESH),
        scratch_types=[
            pltpu.VMEM((EB,), jnp.int32),
            pltpu.VMEM((EB,), jnp.float32),
            pltpu.VMEM((EB,), jnp.int32),
            pltpu.VMEM((EB // 128, 128), jnp.int32),
            pltpu.VMEM((NT + 16,), jnp.int32),
            pltpu.VMEM((NT,), jnp.int32),
        ],
        name="lightgcn_partition",
    )


def _make_layer(final):
    return pl.kernel(
        functools.partial(_layer_body, final),
        out_type=jax.ShapeDtypeStruct((NPAD, D), jnp.float32),
        mesh=plsc.VectorSubcoreMesh(**_MESH),
        scratch_types=[
            pltpu.VMEM((NT + 16,), jnp.int32),
            pltpu.VMEM((NT,), jnp.int32),
            pltpu.VMEM((CAP,), jnp.int32),
            pltpu.VMEM((CAP,), jnp.float32),
            pltpu.VMEM((MCAP,), jnp.int32),
            pltpu.VMEM((MCAP,), jnp.float32),
            pltpu.VMEM((GB,), jnp.int32),
            pltpu.VMEM((GB,), jnp.int32),
            pltpu.VMEM((GB,), jnp.float32),
            pltpu.VMEM((GB, D), jnp.float32),
            pltpu.VMEM((RPT, D), jnp.float32),
        ],
        name="lightgcn_layer_final" if final else "lightgcn_layer",
    )


def kernel(edge_index, edge_weight, user_emb, item_emb):
    src = edge_index[0]
    dst = edge_index[1]
    zi = jnp.zeros((EPAD - E,), jnp.int32)
    packed = jnp.concatenate([src, zi]) | (jnp.concatenate([dst, zi]) << PKBITS)
    pk = packed.reshape(-1, EB)
    wp = jnp.concatenate([edge_weight, jnp.zeros((EPAD - E,), jnp.float32)]).reshape(-1, EB)

    ego0 = jnp.zeros((NPAD, D), jnp.float32)
    ego0 = ego0.at[:USER_N].set(user_emb).at[LPAD:LPAD + ITEM_N].set(item_emb)

    rec, rw, cnt = _make_partition()(pk, wp)
    cntt = cnt.T

    layer = _make_layer(False)
    layer_final = _make_layer(True)
    e1 = layer(rec, rw, cntt, ego0)
    e2 = layer(rec, rw, cntt, e1)
    out = layer_final(rec, rw, cntt, e2, ego0, e1)
    return (out[:USER_N], out[LPAD:LPAD + ITEM_N])
